# Initial kernel scaffold; baseline (speedup 1.0000x reference)
import jax
import jax.numpy as jnp
from jax.experimental import pallas as pl


def kernel(boxes, scores):
    raise NotImplementedError("write your pallas kernel here")


# speedup vs baseline: 4.2631x; 4.2631x over previous
import jax
import jax.numpy as jnp
from jax.experimental import pallas as pl
from jax.experimental.pallas import tpu as pltpu

SCORE_THRESHOLD_ = 0.05
IOU_THRESHOLD_ = 0.65
MAX_DETECTIONS_ = 300
PRE_NMS_TOPK_ = 4096
ROWS_ = 32
LANES_ = 128


def _nms_loop_kernel(x1_ref, y1_ref, x2_ref, y2_ref, s_ref, cls_ref,
                     boxes_ref, ob_ref, os_ref, oc_ref, nv_ref):
    X1 = x1_ref[...]
    Y1 = y1_ref[...]
    X2 = x2_ref[...]
    Y2 = y2_ref[...]
    S = s_ref[...]
    CLf = cls_ref[...].astype(jnp.float32)

    allb = boxes_ref[...]
    span = jnp.max(allb) - jnp.min(allb) + 1.0

    off = CLf * span
    OX1 = X1 + off
    OY1 = Y1 + off
    OX2 = X2 + off
    OY2 = Y2 + off
    AREA = (OX2 - OX1) * (OY2 - OY1)

    iota = (jax.lax.broadcasted_iota(jnp.int32, (ROWS_, LANES_), 0) * LANES_
            + jax.lax.broadcasted_iota(jnp.int32, (ROWS_, LANES_), 1))

    valid0 = (S > 0.0).astype(jnp.float32)
    big = jnp.int32(PRE_NMS_TOPK_)

    def body(t, carry):
        valid, n = carry
        i = jnp.min(jnp.where(valid > 0.0, iota, big))
        keep = i < big
        keepf = keep.astype(jnp.float32)
        ohf = (iota == i).astype(jnp.float32)

        sv = jnp.sum(ohf * S)
        cf = jnp.sum(ohf * CLf)
        ox1 = jnp.sum(ohf * OX1)
        oy1 = jnp.sum(ohf * OY1)
        ox2 = jnp.sum(ohf * OX2)
        oy2 = jnp.sum(ohf * OY2)
        bx1 = jnp.sum(ohf * X1)
        by1 = jnp.sum(ohf * Y1)
        bx2 = jnp.sum(ohf * X2)
        by2 = jnp.sum(ohf * Y2)

        xx1 = jnp.maximum(ox1, OX1)
        yy1 = jnp.maximum(oy1, OY1)
        xx2 = jnp.minimum(ox2, OX2)
        yy2 = jnp.minimum(oy2, OY2)
        inter = jnp.clip(xx2 - xx1, 0.0) * jnp.clip(yy2 - yy1, 0.0)
        a1 = (ox2 - ox1) * (oy2 - oy1)
        iou = inter / (a1 + AREA - inter + 1e-9)
        valid = jnp.where(iou <= IOU_THRESHOLD_, valid, 0.0)

        ob_ref[t, 0] = bx1 * keepf
        ob_ref[t, 1] = by1 * keepf
        ob_ref[t, 2] = bx2 * keepf
        ob_ref[t, 3] = by2 * keepf
        os_ref[t] = sv * keepf
        oc_ref[t] = jnp.where(keep, cf.astype(jnp.int32), jnp.int32(-1))
        return valid, n + keep.astype(jnp.int32)

    _, n = jax.lax.fori_loop(0, MAX_DETECTIONS_, body, (valid0, jnp.int32(0)))
    nv_ref[0] = n


def kernel(boxes, scores):
    b = boxes[0]
    s = scores[0]
    N, C = s.shape
    flat = s.reshape(-1)
    flat = jnp.where(flat >= SCORE_THRESHOLD_, flat, -1.0)
    K = PRE_NMS_TOPK_
    top_scores, top_pos = jax.lax.top_k(flat, K)
    box_idx = top_pos // C
    cls = (top_pos % C).astype(jnp.int32)
    cb = jnp.take(b, box_idx, axis=0)

    X1 = cb[:, 0].reshape(ROWS_, LANES_)
    Y1 = cb[:, 1].reshape(ROWS_, LANES_)
    X2 = cb[:, 2].reshape(ROWS_, LANES_)
    Y2 = cb[:, 3].reshape(ROWS_, LANES_)
    S2 = top_scores.reshape(ROWS_, LANES_)
    CL = cls.reshape(ROWS_, LANES_)
    boxes_flat = b.reshape(-1, LANES_)

    smem = pl.BlockSpec(memory_space=pltpu.SMEM)
    ob, os_, oc, nv = pl.pallas_call(
        _nms_loop_kernel,
        out_shape=(
            jax.ShapeDtypeStruct((MAX_DETECTIONS_, 4), jnp.float32),
            jax.ShapeDtypeStruct((MAX_DETECTIONS_,), jnp.float32),
            jax.ShapeDtypeStruct((MAX_DETECTIONS_,), jnp.int32),
            jax.ShapeDtypeStruct((1,), jnp.int32),
        ),
        out_specs=(smem, smem, smem, smem),
    )(X1, Y1, X2, Y2, S2, CL, boxes_flat)

    return ob[None], os_[None], oc[None], nv[None]
